# Initial kernel scaffold; baseline (speedup 1.0000x reference)
import jax
import jax.numpy as jnp
from jax.experimental import pallas as pl


def kernel(x, src0, dst0, src1, dst1, src2, dst2, Wl0, Wr0, b0, Wl1, Wr1, b1, Wl2, Wr2, b2):
    raise NotImplementedError("write your pallas kernel here")


# speedup vs baseline: 2.5221x; 2.5221x over previous
import functools

import jax
import jax.numpy as jnp
from jax import lax
from jax.experimental import pallas as pl
from jax.experimental.pallas import tpu as pltpu
from jax.experimental.pallas import tpu_sc as plsc

_NC = 2
_NS = 16
_NW = _NC * _NS
_C = 128


def _make_seg_kernel(n_pad, nb, blk, bsz):
    mesh = plsc.VectorSubcoreMesh(core_axis_name="c", subcore_axis_name="s")

    @functools.partial(
        pl.kernel,
        mesh=mesh,
        out_type=(
            jax.ShapeDtypeStruct((n_pad, 128), jnp.float32),
            jax.ShapeDtypeStruct((n_pad, 16), jnp.float32),
        ),
        scratch_types=[
            pltpu.VMEM((bsz,), jnp.int32),
            pltpu.VMEM((_C,), jnp.int32),
            pltpu.VMEM((_C + 16,), jnp.int32),
            pltpu.VMEM((_C, 128), jnp.float32),
            pltpu.VMEM((blk + 1, 128), jnp.float32),
            pltpu.VMEM((blk + 1, 16), jnp.float32),
            pltpu.SemaphoreType.DMA,
        ],
    )
    def seg(h_hbm, src_hbm, dst_hbm, bounds_hbm, sum_hbm, cnt_hbm,
            bounds_v, sidx_v, dval_v, rows_v, acc_v, cnt_v, sem):
        cid = lax.axis_index("c")
        sid = lax.axis_index("s")
        wid = sid * _NC + cid
        pltpu.sync_copy(bounds_hbm, bounds_v)

        zero16 = jnp.zeros((16,), jnp.float32)
        ones16 = jnp.ones((16,), jnp.float32)

        def block(b, carry0):
            g = wid * nb + b
            dst_lo = g * blk
            bvec = bounds_v[pl.ds(g, 16)]
            e_lo = bvec[0]
            e_hi = bvec[1]
            base = (e_lo // 8) * 8
            nchunks = (e_hi - base + (_C - 1)) // _C

            def zrow(i, carry):
                for l in range(8):
                    acc_v[i, pl.ds(16 * l, 16)] = zero16
                cnt_v[i, :] = zero16
                return carry

            lax.fori_loop(0, blk + 1, zrow, 0)

            def chunk(k, carry):
                e0 = base + k * _C
                pltpu.sync_copy(src_hbm.at[pl.ds(e0, _C)], sidx_v)
                pltpu.sync_copy(dst_hbm.at[pl.ds(e0, _C)],
                                dval_v.at[pl.ds(0, _C)])
                pltpu.async_copy(h_hbm.at[sidx_v], rows_v, sem).wait()

                def edge(j, cc):
                    dl = dval_v[pl.ds(j, 16)][0] - dst_lo
                    valid = (dl >= 0) & (dl < blk)
                    dl = jnp.where(valid, dl, blk)
                    for l in range(8):
                        sl = pl.ds(16 * l, 16)
                        acc_v[dl, sl] = acc_v[dl, sl] + rows_v[j, sl]
                    cnt_v[dl, :] = cnt_v[dl, :] + ones16
                    return cc

                lax.fori_loop(0, _C, edge, 0)
                return carry

            lax.fori_loop(0, nchunks, chunk, 0)

            pltpu.sync_copy(acc_v.at[pl.ds(0, blk)],
                            sum_hbm.at[pl.ds(dst_lo, blk)])
            pltpu.sync_copy(cnt_v.at[pl.ds(0, blk)],
                            cnt_hbm.at[pl.ds(dst_lo, blk)])
            return carry0

        lax.fori_loop(0, nb, block, 0)

    return seg


def _tc_block(sum_ref, cnt_ref, xt_ref, wl_ref, wr_ref, b_ref, out_ref, *, relu):
    cnt = cnt_ref[:, 0:1]
    mean = sum_ref[:] / jnp.maximum(cnt, 1.0)
    acc = jnp.dot(mean, wl_ref[:], preferred_element_type=jnp.float32)
    acc = acc + jnp.dot(xt_ref[:], wr_ref[:], preferred_element_type=jnp.float32)
    acc = acc + b_ref[:]
    out_ref[:] = jnp.maximum(acc, 0.0) if relu else acc


def _sage_dense(summed, cnt16, x_tgt, Wl, Wr, b, relu, blk):
    n = summed.shape[0]
    return pl.pallas_call(
        functools.partial(_tc_block, relu=relu),
        grid=(n // blk,),
        in_specs=[
            pl.BlockSpec((blk, 128), lambda i: (i, 0)),
            pl.BlockSpec((blk, 16), lambda i: (i, 0)),
            pl.BlockSpec((blk, 128), lambda i: (i, 0)),
            pl.BlockSpec((128, 128), lambda i: (0, 0)),
            pl.BlockSpec((128, 128), lambda i: (0, 0)),
            pl.BlockSpec((1, 128), lambda i: (0, 0)),
        ],
        out_specs=pl.BlockSpec((blk, 128), lambda i: (i, 0)),
        out_shape=jax.ShapeDtypeStruct((n, 128), jnp.float32),
    )(summed, cnt16, x_tgt, Wl, Wr, b)


_CFG = {20000: (4, 1000), 8000: (2, 1000), 4096: (1, 512)}


def _layer(h, src, dst, n_tgt, Wl, Wr, b, relu):
    e = src.shape[0]
    nb, tc_blk = _CFG[n_tgt]
    blk = -(-n_tgt // (_NW * nb * 8)) * 8
    n_pad = blk * _NW * nb
    ng = _NW * nb
    bsz = ((ng + 16 + 7) // 8) * 8
    boundaries = jnp.arange(ng + 1, dtype=jnp.int32) * blk
    ebounds = jnp.searchsorted(dst.astype(jnp.int32), boundaries, side="left")
    ebounds = jnp.pad(ebounds.astype(jnp.int32), (0, bsz - (ng + 1)),
                      constant_values=e)
    src_p = jnp.pad(src.astype(jnp.int32), (0, _C))
    dst_p = jnp.pad(dst.astype(jnp.int32), (0, _C), constant_values=n_pad)
    summed, cnt16 = _make_seg_kernel(n_pad, nb, blk, bsz)(h, src_p, dst_p, ebounds)
    return _sage_dense(summed[:n_tgt], cnt16[:n_tgt], h[:n_tgt], Wl, Wr,
                       b.reshape(1, 128), relu, tc_blk)


def kernel(x, src0, dst0, src1, dst1, src2, dst2,
           Wl0, Wr0, b0, Wl1, Wr1, b1, Wl2, Wr2, b2):
    h = _layer(x, src0, dst0, 20000, Wl0, Wr0, b0, True)
    h = _layer(h, src1, dst1, 8000, Wl1, Wr1, b1, False)
    h = _layer(h, src2, dst2, 4096, Wl2, Wr2, b2, True)
    return h
